# Initial kernel scaffold; baseline (speedup 1.0000x reference)
import jax
import jax.numpy as jnp
from jax.experimental import pallas as pl


def kernel(x, W, b):
    raise NotImplementedError("write your pallas kernel here")


# speedup vs baseline: 1.1513x; 1.1513x over previous
import jax
import jax.numpy as jnp
from jax.experimental import pallas as pl
from jax.experimental.pallas import tpu as pltpu

_TOPK = 8
_E = 64
_M_BLK = 512


def _gating_block(x_ref, w_ref, b_ref, vals_ref, idx_ref, scores_ref):
    logits = jax.lax.dot_general(
        x_ref[...], w_ref[...], (((1,), (1,)), ((), ())),
        preferred_element_type=jnp.float32)
    logits = logits + b_ref[...]
    m = jnp.max(logits, axis=-1, keepdims=True)
    e = jnp.exp(logits - m)
    s = jnp.sum(e, axis=-1, keepdims=True)
    scores = e / s
    scores_ref[...] = scores

    col = jax.lax.broadcasted_iota(jnp.int32, scores.shape, 1)
    work = scores
    vals, idxs = [], []
    for _ in range(_TOPK):
        mx = jnp.max(work, axis=-1, keepdims=True)
        amx = jnp.min(jnp.where(work == mx, col, _E), axis=-1, keepdims=True)
        vals.append(mx)
        idxs.append(amx)
        work = jnp.where(col == amx, -1.0, work)
    vals_ref[...] = jnp.concatenate(vals, axis=1)
    idx_ref[...] = jnp.concatenate(idxs, axis=1)


def kernel(x, W, b):
    n_tokens, d_model = x.shape
    n_exp = W.shape[0]
    b2 = b.reshape(1, n_exp)
    grid = (n_tokens // _M_BLK,)
    vals, idx, scores = pl.pallas_call(
        _gating_block,
        grid=grid,
        in_specs=[
            pl.BlockSpec((_M_BLK, d_model), lambda i: (i, 0)),
            pl.BlockSpec((n_exp, d_model), lambda i: (0, 0)),
            pl.BlockSpec((1, n_exp), lambda i: (0, 0)),
        ],
        out_specs=[
            pl.BlockSpec((_M_BLK, _TOPK), lambda i: (i, 0)),
            pl.BlockSpec((_M_BLK, _TOPK), lambda i: (i, 0)),
            pl.BlockSpec((_M_BLK, n_exp), lambda i: (i, 0)),
        ],
        out_shape=[
            jax.ShapeDtypeStruct((n_tokens, _TOPK), jnp.float32),
            jax.ShapeDtypeStruct((n_tokens, _TOPK), jnp.int32),
            jax.ShapeDtypeStruct((n_tokens, n_exp), jnp.float32),
        ],
    )(x, W, b2)
    return (vals, idx, scores)


# speedup vs baseline: 1.3012x; 1.1302x over previous
import jax
import jax.numpy as jnp
from jax.experimental import pallas as pl
from jax.experimental.pallas import tpu as pltpu

_TOPK = 8
_E = 64
_M_BLK = 1024


def _gating_block(x_ref, w_ref, b_ref, vals_ref, idx_ref, scores_ref):
    logits = jax.lax.dot_general(
        x_ref[...], w_ref[...], (((1,), (1,)), ((), ())),
        preferred_element_type=jnp.float32)
    logits = logits + b_ref[...]
    m = jnp.max(logits, axis=-1, keepdims=True)
    e = jnp.exp(logits - m)
    s = jnp.sum(e, axis=-1, keepdims=True)
    scores = e / s
    scores_ref[...] = scores

    col = jax.lax.broadcasted_iota(jnp.int32, scores.shape, 1)
    work = scores
    vals, idxs = [], []
    for _ in range(_TOPK):
        mx = jnp.max(work, axis=-1, keepdims=True)
        amx = jnp.min(jnp.where(work == mx, col, _E), axis=-1, keepdims=True)
        vals.append(mx)
        idxs.append(amx)
        work = jnp.where(col == amx, -1.0, work)
    vals_ref[...] = jnp.concatenate(vals, axis=1)
    idx_ref[...] = jnp.concatenate(idxs, axis=1)


def kernel(x, W, b):
    n_tokens, d_model = x.shape
    n_exp = W.shape[0]
    b2 = b.reshape(1, n_exp)
    grid = (n_tokens // _M_BLK,)
    vals, idx, scores = pl.pallas_call(
        _gating_block,
        grid=grid,
        in_specs=[
            pl.BlockSpec((_M_BLK, d_model), lambda i: (i, 0)),
            pl.BlockSpec((n_exp, d_model), lambda i: (0, 0)),
            pl.BlockSpec((1, n_exp), lambda i: (0, 0)),
        ],
        out_specs=[
            pl.BlockSpec((_M_BLK, _TOPK), lambda i: (i, 0)),
            pl.BlockSpec((_M_BLK, _TOPK), lambda i: (i, 0)),
            pl.BlockSpec((_M_BLK, n_exp), lambda i: (i, 0)),
        ],
        out_shape=[
            jax.ShapeDtypeStruct((n_tokens, _TOPK), jnp.float32),
            jax.ShapeDtypeStruct((n_tokens, _TOPK), jnp.int32),
            jax.ShapeDtypeStruct((n_tokens, n_exp), jnp.float32),
        ],
    )(x, W, b2)
    return (vals, idx, scores)


# speedup vs baseline: 1.4633x; 1.1246x over previous
import jax
import jax.numpy as jnp
from jax.experimental import pallas as pl
from jax.experimental.pallas import tpu as pltpu

_TOPK = 8
_E = 64
_M_BLK = 1024


def _gating_block(x_ref, w_ref, b_ref, vals_ref, idx_ref, scores_ref):
    logits_t = jax.lax.dot_general(
        w_ref[...], x_ref[...], (((1,), (1,)), ((), ())),
        preferred_element_type=jnp.float32)
    logits_t = logits_t + b_ref[...][:, 0:1]
    m = jnp.max(logits_t, axis=0, keepdims=True)
    e = jnp.exp(logits_t - m)
    s = jnp.sum(e, axis=0, keepdims=True)
    scores_t = e / s
    scores_ref[...] = scores_t.T

    row = jax.lax.broadcasted_iota(jnp.int32, scores_t.shape, 0)
    work = scores_t
    vals, idxs = [], []
    for _ in range(_TOPK):
        mx = jnp.max(work, axis=0, keepdims=True)
        amx = jnp.min(jnp.where(work == mx, row, _E), axis=0, keepdims=True)
        vals.append(mx)
        idxs.append(amx)
        work = jnp.where(row == amx, -1.0, work)
    vals_ref[...] = jnp.concatenate(vals, axis=0).T
    idx_ref[...] = jnp.concatenate(idxs, axis=0).T


def kernel(x, W, b):
    n_tokens, d_model = x.shape
    n_exp = W.shape[0]
    b2 = jnp.broadcast_to(b.reshape(n_exp, 1), (n_exp, 128))
    grid = (n_tokens // _M_BLK,)
    vals, idx, scores = pl.pallas_call(
        _gating_block,
        grid=grid,
        in_specs=[
            pl.BlockSpec((_M_BLK, d_model), lambda i: (i, 0)),
            pl.BlockSpec((n_exp, d_model), lambda i: (0, 0)),
            pl.BlockSpec((n_exp, 128), lambda i: (0, 0)),
        ],
        out_specs=[
            pl.BlockSpec((_M_BLK, _TOPK), lambda i: (i, 0)),
            pl.BlockSpec((_M_BLK, _TOPK), lambda i: (i, 0)),
            pl.BlockSpec((_M_BLK, n_exp), lambda i: (i, 0)),
        ],
        out_shape=[
            jax.ShapeDtypeStruct((n_tokens, _TOPK), jnp.float32),
            jax.ShapeDtypeStruct((n_tokens, _TOPK), jnp.int32),
            jax.ShapeDtypeStruct((n_tokens, n_exp), jnp.float32),
        ],
    )(x, W, b2)
    return (vals, idx, scores)
